# initial kernel scaffold (unmeasured)
import jax
import jax.numpy as jnp
from jax import lax
from jax.experimental import pallas as pl
from jax.experimental.pallas import tpu as pltpu

N_DEV = 8
SQ = 2048
SKV = 2048
HQ = 8
DH = 128
DM = 1024
BLK = 64
QCHUNK = 512
SCALE = 0.08838834764831843
NEG = -1e9


def kernel(x, Wq, K_ext, V_ext, Wo):
    def body(x_ref, wq_ref, k_ref, v_ref, wo_ref, out_ref,
             ctx_acc, ml_acc, comm_ctx, comm_ml,
             ctx_send, ctx_recv, ml_send, ml_recv):
        my = lax.axis_index("i")
        left = lax.rem(my + N_DEV - 1, N_DEV)
        right = lax.rem(my + 1, N_DEV)

        barrier = pltpu.get_barrier_semaphore()
        for nbr in (left, right):
            pl.semaphore_signal(barrier, inc=1, device_id=(nbr,),
                                device_id_type=pl.DeviceIdType.MESH)
        pl.semaphore_wait(barrier, 2)

        xq = x_ref[0].astype(jnp.bfloat16)
        wq = wq_ref[...].astype(jnp.bfloat16)
        q = lax.dot_general(xq, wq, (((1,), (0,)), ((), ())),
                            preferred_element_type=jnp.float32)
        q = (q * SCALE).astype(jnp.bfloat16)
        k = k_ref[0].astype(jnp.bfloat16)
        v = v_ref[0].astype(jnp.bfloat16)

        kb_off = my * (SKV // BLK)
        for qc in range(SQ // QCHUNK):
            r0 = qc * QCHUNK
            qb = (lax.broadcasted_iota(jnp.int32, (QCHUNK, SKV), 0) + r0) // BLK
            kb = lax.broadcasted_iota(jnp.int32, (QCHUNK, SKV), 1) // BLK + kb_off
            active = (qb == kb) | (kb == 0) | (lax.rem(qb + kb, 3) == 0)
            bias = jnp.where(active, 0.0, NEG).astype(jnp.float32)
            for h in range(HQ):
                qh = q[r0:r0 + QCHUNK, h * DH:(h + 1) * DH]
                s = lax.dot_general(qh, k[:, h, :], (((1,), (1,)), ((), ())),
                                    preferred_element_type=jnp.float32) + bias
                m = jnp.max(s, axis=1, keepdims=True)
                p = jnp.exp(s - m)
                lsum = jnp.sum(p, axis=1, keepdims=True)
                ctxh = lax.dot_general(p.astype(jnp.bfloat16), v[:, h, :],
                                       (((1,), (0,)), ((), ())),
                                       preferred_element_type=jnp.float32)
                ctx_acc[r0:r0 + QCHUNK, h * DH:(h + 1) * DH] = ctxh
                ml_acc[0, r0:r0 + QCHUNK, h:h + 1] = m
                ml_acc[1, r0:r0 + QCHUNK, h:h + 1] = lsum

        comm_ctx[0] = ctx_acc[...]
        comm_ml[0] = ml_acc[...]

        for hop in range(N_DEV - 1):
            s_slot, r_slot = hop % 2, (hop + 1) % 2
            rc = pltpu.make_async_remote_copy(
                src_ref=comm_ctx.at[s_slot], dst_ref=comm_ctx.at[r_slot],
                send_sem=ctx_send.at[hop], recv_sem=ctx_recv.at[hop],
                device_id=(right,), device_id_type=pl.DeviceIdType.MESH)
            rm = pltpu.make_async_remote_copy(
                src_ref=comm_ml.at[s_slot], dst_ref=comm_ml.at[r_slot],
                send_sem=ml_send.at[hop], recv_sem=ml_recv.at[hop],
                device_id=(right,), device_id_type=pl.DeviceIdType.MESH)
            rc.start()
            rm.start()
            rc.wait()
            rm.wait()

            m_in = comm_ml[r_slot, 0]
            l_in = comm_ml[r_slot, 1]
            m_old = ml_acc[0]
            m_new = jnp.maximum(m_old, m_in)
            a = jnp.exp(m_old - m_new)
            b = jnp.exp(m_in - m_new)
            ml_acc[0] = m_new
            ml_acc[1] = a * ml_acc[1] + b * l_in
            for h in range(HQ):
                sl = slice(h * DH, (h + 1) * DH)
                ctx_acc[:, sl] = (a[:, h:h + 1] * ctx_acc[:, sl]
                                  + b[:, h:h + 1] * comm_ctx[r_slot][:, sl])

        l_fin = ml_acc[1]
        for h in range(HQ):
            sl = slice(h * DH, (h + 1) * DH)
            ctx_acc[:, sl] = ctx_acc[:, sl] / l_fin[:, h:h + 1]
        wo = wo_ref[...].astype(jnp.bfloat16)
        out_ref[0] = lax.dot_general(ctx_acc[...].astype(jnp.bfloat16), wo,
                                     (((1,), (0,)), ((), ())),
                                     preferred_element_type=jnp.float32)

    return pl.pallas_call(
        body,
        out_shape=jax.ShapeDtypeStruct((1, SQ, DM), jnp.float32),
        in_specs=[pl.BlockSpec(memory_space=pltpu.VMEM)] * 5,
        out_specs=pl.BlockSpec(memory_space=pltpu.VMEM),
        scratch_shapes=[
            pltpu.VMEM((SQ, DM), jnp.float32),
            pltpu.VMEM((2, SQ, HQ), jnp.float32),
            pltpu.VMEM((2, SQ, DM), jnp.float32),
            pltpu.VMEM((2, 2, SQ, HQ), jnp.float32),
            pltpu.SemaphoreType.DMA((N_DEV - 1,)),
            pltpu.SemaphoreType.DMA((N_DEV - 1,)),
            pltpu.SemaphoreType.DMA((N_DEV - 1,)),
            pltpu.SemaphoreType.DMA((N_DEV - 1,)),
        ],
        compiler_params=pltpu.CompilerParams(collective_id=0),
    )(x, Wq, K_ext, V_ext, Wo)


# baseline (device time: 655497 ns/iter reference)
import jax
import jax.numpy as jnp
from jax import lax
from jax.experimental import pallas as pl
from jax.experimental.pallas import tpu as pltpu

N_DEV = 8
SQ = 2048
SKV = 2048
HQ = 8
DH = 128
DM = 1024
BLK = 64
QCHUNK = 256
SCALE = 0.08838834764831843
NEG = -1e9


def kernel(x, Wq, K_ext, V_ext, Wo):
    def body(x_ref, wq_ref, kt_ref, vt_ref, wo_ref, out_ref,
             q_buf, ctx_acc, ml_acc, comm_ctx, comm_ml,
             ctx_send, ctx_recv, ml_send, ml_recv):
        my = lax.axis_index("i")
        left = lax.rem(my + N_DEV - 1, N_DEV)
        right = lax.rem(my + 1, N_DEV)

        barrier = pltpu.get_barrier_semaphore()
        for nbr in (left, right):
            pl.semaphore_signal(barrier, inc=1, device_id=(nbr,),
                                device_id_type=pl.DeviceIdType.MESH)
        pl.semaphore_wait(barrier, 2)

        def qproj_step(c, carry):
            r0 = c * QCHUNK
            q_buf[pl.ds(r0, QCHUNK), :] = (lax.dot_general(
                x_ref[0, pl.ds(r0, QCHUNK), :], wq_ref[...],
                (((1,), (0,)), ((), ())),
                preferred_element_type=jnp.float32) * SCALE).astype(jnp.bfloat16)
            return carry

        lax.fori_loop(0, SQ // QCHUNK, qproj_step, 0)

        kb_off = my * (SKV // BLK)

        def qc_step(qc, carry):
            r0 = qc * QCHUNK
            qb = (lax.broadcasted_iota(jnp.int32, (QCHUNK, SKV), 0) + r0) // BLK
            kb = lax.broadcasted_iota(jnp.int32, (QCHUNK, SKV), 1) // BLK + kb_off
            active = (qb == kb) | (kb == 0) | (lax.rem(qb + kb, 3) == 0)
            bias = jnp.where(active, 0.0, NEG).astype(jnp.float32)

            for h in range(HQ):
                qh = q_buf[pl.ds(r0, QCHUNK), h * DH:(h + 1) * DH]
                kh = kt_ref[h]
                vh = vt_ref[h]
                s = lax.dot_general(qh, kh, (((1,), (1,)), ((), ())),
                                    preferred_element_type=jnp.float32) + bias
                m = jnp.max(s, axis=1, keepdims=True)
                p = jnp.exp(s - m)
                lsum = jnp.sum(p, axis=1, keepdims=True)
                ctxh = lax.dot_general(p.astype(jnp.bfloat16), vh,
                                       (((1,), (0,)), ((), ())),
                                       preferred_element_type=jnp.float32)
                ctx_acc[pl.ds(r0, QCHUNK), h, :] = ctxh
                ml_acc[0, pl.ds(r0, QCHUNK), h:h + 1] = m
                ml_acc[1, pl.ds(r0, QCHUNK), h:h + 1] = lsum
            return carry

        lax.fori_loop(0, SQ // QCHUNK, qc_step, 0)

        comm_ctx[0] = ctx_acc[...].astype(jnp.bfloat16)
        comm_ml[0] = ml_acc[...]

        for hop in range(N_DEV - 1):
            s_slot, r_slot = hop % 2, (hop + 1) % 2
            rc = pltpu.make_async_remote_copy(
                src_ref=comm_ctx.at[s_slot], dst_ref=comm_ctx.at[r_slot],
                send_sem=ctx_send.at[hop], recv_sem=ctx_recv.at[hop],
                device_id=(right,), device_id_type=pl.DeviceIdType.MESH)
            rm = pltpu.make_async_remote_copy(
                src_ref=comm_ml.at[s_slot], dst_ref=comm_ml.at[r_slot],
                send_sem=ml_send.at[hop], recv_sem=ml_recv.at[hop],
                device_id=(right,), device_id_type=pl.DeviceIdType.MESH)
            rc.start()
            rm.start()
            rc.wait()
            rm.wait()

            def merge_step(c, carry):
                r0 = c * QCHUNK
                rs = pl.ds(r0, QCHUNK)
                m_in = comm_ml[r_slot, 0, rs, :]
                l_in = comm_ml[r_slot, 1, rs, :]
                m_old = ml_acc[0, rs, :]
                m_new = jnp.maximum(m_old, m_in)
                a = jnp.exp(m_old - m_new)
                b = jnp.exp(m_in - m_new)
                ml_acc[0, rs, :] = m_new
                ml_acc[1, rs, :] = a * ml_acc[1, rs, :] + b * l_in
                ctx_acc[rs, :, :] = (
                    a[:, :, None] * ctx_acc[rs, :, :]
                    + b[:, :, None] * comm_ctx[r_slot, rs, :, :].astype(jnp.float32))
                return carry

            lax.fori_loop(0, SQ // QCHUNK, merge_step, 0)

        def proj_step(c, carry):
            r0 = c * QCHUNK
            rs = pl.ds(r0, QCHUNK)
            l_fin = ml_acc[1, rs, :]
            acc = jnp.zeros((QCHUNK, DM), jnp.float32)
            for h in range(HQ):
                ch = (ctx_acc[rs, h, :] / l_fin[:, h:h + 1]).astype(jnp.bfloat16)
                acc = acc + lax.dot_general(
                    ch, wo_ref[h * DH:(h + 1) * DH, :], (((1,), (0,)), ((), ())),
                    preferred_element_type=jnp.float32)
            out_ref[0, rs, :] = acc
            return carry

        lax.fori_loop(0, SQ // QCHUNK, proj_step, 0)

    call = pl.pallas_call(
        body,
        out_shape=jax.ShapeDtypeStruct((1, SQ, DM), jnp.float32),
        in_specs=[pl.BlockSpec(memory_space=pltpu.VMEM)] * 5,
        out_specs=pl.BlockSpec(memory_space=pltpu.VMEM),
        scratch_shapes=[
            pltpu.VMEM((SQ, DM), jnp.bfloat16),
            pltpu.VMEM((SQ, HQ, DH), jnp.float32),
            pltpu.VMEM((2, SQ, HQ), jnp.float32),
            pltpu.VMEM((2, SQ, HQ, DH), jnp.bfloat16),
            pltpu.VMEM((2, 2, SQ, HQ), jnp.float32),
            pltpu.SemaphoreType.DMA((N_DEV - 1,)),
            pltpu.SemaphoreType.DMA((N_DEV - 1,)),
            pltpu.SemaphoreType.DMA((N_DEV - 1,)),
            pltpu.SemaphoreType.DMA((N_DEV - 1,)),
        ],
        compiler_params=pltpu.CompilerParams(
            collective_id=0, vmem_limit_bytes=63 * 1024 * 1024),
    )
    kt = jnp.transpose(K_ext[0].astype(jnp.bfloat16), (1, 0, 2))
    vt = jnp.transpose(V_ext[0].astype(jnp.bfloat16), (1, 0, 2))
    return call(x.astype(jnp.bfloat16), Wq.astype(jnp.bfloat16),
                kt, vt, Wo.astype(jnp.bfloat16))


# device time: 255220 ns/iter; 2.5684x vs baseline; 2.5684x over previous
import jax
import jax.numpy as jnp
from jax import lax
from jax.experimental import pallas as pl
from jax.experimental.pallas import tpu as pltpu

N_DEV = 8
SQ = 2048
SKV = 2048
HQ = 8
DH = 128
DM = 1024
BLK = 64
CHUNK = SQ // N_DEV
SCALE = 0.08838834764831843
NEG = -1e9


def kernel(x, Wq, K_ext, V_ext, Wo):
    def body(x_ref, wq_ref, kt_ref, vt_ref, wo_ref, out_ref,
             q_buf, ctx_buf, ml_buf, recv_ctx, recv_ml,
             rs_send, rs_recv, ml_send, ml_recv, ag_send, ag_recv):
        my = lax.axis_index("i")
        left = lax.rem(my + N_DEV - 1, N_DEV)
        right = lax.rem(my + 1, N_DEV)

        barrier = pltpu.get_barrier_semaphore()
        for nbr in (left, right):
            pl.semaphore_signal(barrier, inc=1, device_id=(nbr,),
                                device_id_type=pl.DeviceIdType.MESH)
        pl.semaphore_wait(barrier, 2)

        def qproj_step(c, carry):
            r0 = c * CHUNK
            q_buf[pl.ds(r0, CHUNK), :] = (lax.dot_general(
                x_ref[0, pl.ds(r0, CHUNK), :], wq_ref[...],
                (((1,), (0,)), ((), ())),
                preferred_element_type=jnp.float32) * SCALE).astype(jnp.bfloat16)
            return carry

        lax.fori_loop(0, N_DEV, qproj_step, 0)

        kb_off = my * (SKV // BLK)

        def compute_chunk(r0):
            qb = (lax.broadcasted_iota(jnp.int32, (CHUNK, SKV), 0) + r0) // BLK
            kb = lax.broadcasted_iota(jnp.int32, (CHUNK, SKV), 1) // BLK + kb_off
            active = (qb == kb) | (kb == 0) | (lax.rem(qb + kb, 3) == 0)
            bias = jnp.where(active, 0.0, NEG).astype(jnp.float32)
            for h in range(HQ):
                qh = q_buf[pl.ds(r0, CHUNK), h * DH:(h + 1) * DH]
                s = lax.dot_general(qh, kt_ref[h], (((1,), (1,)), ((), ())),
                                    preferred_element_type=jnp.float32) + bias
                m = jnp.max(s, axis=1, keepdims=True)
                p = jnp.exp(s - m)
                lsum = jnp.sum(p, axis=1, keepdims=True)
                ctxh = lax.dot_general(p.astype(jnp.bfloat16), vt_ref[h],
                                       (((1,), (0,)), ((), ())),
                                       preferred_element_type=jnp.float32)
                ctx_buf[pl.ds(r0, CHUNK), h, :] = ctxh
                ml_buf[pl.ds(r0, CHUNK), h:h + 1] = m
                ml_buf[pl.ds(r0, CHUNK), HQ + h:HQ + h + 1] = lsum

        def merge_recv(r0, slot):
            rs = pl.ds(r0, CHUNK)
            m_loc = ml_buf[rs, 0:HQ]
            l_loc = ml_buf[rs, HQ:2 * HQ]
            m_in = recv_ml[slot, :, 0:HQ]
            l_in = recv_ml[slot, :, HQ:2 * HQ]
            m_new = jnp.maximum(m_loc, m_in)
            a = jnp.exp(m_loc - m_new)
            b = jnp.exp(m_in - m_new)
            ml_buf[rs, 0:HQ] = m_new
            ml_buf[rs, HQ:2 * HQ] = a * l_loc + b * l_in
            ctx_buf[rs, :, :] = (a[:, :, None] * ctx_buf[rs, :, :]
                                 + b[:, :, None] * recv_ctx[slot])

        pending = []
        for t in range(N_DEV):
            c = lax.rem(my - t + N_DEV, N_DEV)
            r0 = c * CHUNK
            compute_chunk(r0)
            if t > 0:
                pending[2 * (t - 1)].wait_recv()
                pending[2 * (t - 1) + 1].wait_recv()
                merge_recv(r0, t - 1)
            if t < N_DEV - 1:
                rc = pltpu.make_async_remote_copy(
                    src_ref=ctx_buf.at[pl.ds(r0, CHUNK)],
                    dst_ref=recv_ctx.at[t],
                    send_sem=rs_send.at[t], recv_sem=rs_recv.at[t],
                    device_id=(right,), device_id_type=pl.DeviceIdType.MESH)
                rm = pltpu.make_async_remote_copy(
                    src_ref=ml_buf.at[pl.ds(r0, CHUNK)],
                    dst_ref=recv_ml.at[t],
                    send_sem=ml_send.at[t], recv_sem=ml_recv.at[t],
                    device_id=(right,), device_id_type=pl.DeviceIdType.MESH)
                rc.start()
                rm.start()
                pending.extend([rc, rm])

        own = lax.rem(my + 1, N_DEV)
        r0o = own * CHUNK
        rso = pl.ds(r0o, CHUNK)
        l_fin = ml_buf[rso, HQ:2 * HQ]
        acc = jnp.zeros((CHUNK, DM), jnp.float32)
        for h in range(HQ):
            ch = (ctx_buf[rso, h, :] / l_fin[:, h:h + 1]).astype(jnp.bfloat16)
            acc = acc + lax.dot_general(
                ch, wo_ref[h * DH:(h + 1) * DH, :], (((1,), (0,)), ((), ())),
                preferred_element_type=jnp.float32)
        out_ref[0, rso, :] = acc

        prev = None
        for u in range(N_DEV - 1):
            g = lax.rem(my + 1 - u + N_DEV, N_DEV)
            rg = pl.ds(g * CHUNK, CHUNK)
            if prev is not None:
                prev.wait_recv()
            ag = pltpu.make_async_remote_copy(
                src_ref=out_ref.at[0, rg],
                dst_ref=out_ref.at[0, rg],
                send_sem=ag_send.at[u], recv_sem=ag_recv.at[u],
                device_id=(right,), device_id_type=pl.DeviceIdType.MESH)
            ag.start()
            pending.append(ag)
            prev = ag
        prev.wait_recv()

        for r in pending:
            r.wait_send()

    call = pl.pallas_call(
        body,
        out_shape=jax.ShapeDtypeStruct((1, SQ, DM), jnp.float32),
        in_specs=[pl.BlockSpec(memory_space=pltpu.VMEM)] * 5,
        out_specs=pl.BlockSpec(memory_space=pltpu.VMEM),
        scratch_shapes=[
            pltpu.VMEM((SQ, DM), jnp.bfloat16),
            pltpu.VMEM((SQ, HQ, DH), jnp.float32),
            pltpu.VMEM((SQ, 2 * HQ), jnp.float32),
            pltpu.VMEM((N_DEV - 1, CHUNK, HQ, DH), jnp.float32),
            pltpu.VMEM((N_DEV - 1, CHUNK, 2 * HQ), jnp.float32),
            pltpu.SemaphoreType.DMA((N_DEV - 1,)),
            pltpu.SemaphoreType.DMA((N_DEV - 1,)),
            pltpu.SemaphoreType.DMA((N_DEV - 1,)),
            pltpu.SemaphoreType.DMA((N_DEV - 1,)),
            pltpu.SemaphoreType.DMA((N_DEV - 1,)),
            pltpu.SemaphoreType.DMA((N_DEV - 1,)),
        ],
        compiler_params=pltpu.CompilerParams(
            collective_id=0, vmem_limit_bytes=63 * 1024 * 1024),
    )
    kt = jnp.transpose(K_ext[0].astype(jnp.bfloat16), (1, 0, 2))
    vt = jnp.transpose(V_ext[0].astype(jnp.bfloat16), (1, 0, 2))
    return call(x.astype(jnp.bfloat16), Wq.astype(jnp.bfloat16),
                kt, vt, Wo.astype(jnp.bfloat16))


# device time: 217421 ns/iter; 3.0149x vs baseline; 1.1739x over previous
import jax
import jax.numpy as jnp
from jax import lax
from jax.experimental import pallas as pl
from jax.experimental.pallas import tpu as pltpu

N_DEV = 8
SQ = 2048
SKV = 2048
HQ = 8
DH = 128
DM = 1024
BLK = 64
CHUNK = SQ // N_DEV
SCALE = 0.08838834764831843
NEG = -1e9


def kernel(x, Wq, K_ext, V_ext, Wo):
    def body(x_ref, wq_ref, kt_ref, vt_ref, wo_ref, out_ref,
             q_buf, ctx_buf, ml_buf, recv_ctx, recv_ml,
             rs_send, rs_recv, ml_send, ml_recv, ag_send, ag_recv,
             agl_send, agl_recv):
        my = lax.axis_index("i")
        left = lax.rem(my + N_DEV - 1, N_DEV)
        right = lax.rem(my + 1, N_DEV)

        barrier = pltpu.get_barrier_semaphore()
        for nbr in (left, right):
            pl.semaphore_signal(barrier, inc=1, device_id=(nbr,),
                                device_id_type=pl.DeviceIdType.MESH)
        pl.semaphore_wait(barrier, 2)

        def qproj_step(c, carry):
            r0 = c * CHUNK
            q_buf[pl.ds(r0, CHUNK), :] = (lax.dot_general(
                x_ref[0, pl.ds(r0, CHUNK), :], wq_ref[...],
                (((1,), (0,)), ((), ())),
                preferred_element_type=jnp.float32) * SCALE).astype(jnp.bfloat16)
            return carry

        lax.fori_loop(0, N_DEV, qproj_step, 0)

        kb_off = my * (SKV // BLK)

        def compute_chunk(r0):
            qb = (lax.broadcasted_iota(jnp.int32, (CHUNK, SKV), 0) + r0) // BLK
            kb = lax.broadcasted_iota(jnp.int32, (CHUNK, SKV), 1) // BLK + kb_off
            active = (qb == kb) | (kb == 0) | (lax.rem(qb + kb, 3) == 0)
            bias = jnp.where(active, 0.0, NEG).astype(jnp.float32)
            for h in range(HQ):
                qh = q_buf[pl.ds(r0, CHUNK), h * DH:(h + 1) * DH]
                s = lax.dot_general(qh, kt_ref[h], (((1,), (1,)), ((), ())),
                                    preferred_element_type=jnp.float32) + bias
                m = jnp.max(s, axis=1, keepdims=True)
                p = jnp.exp(s - m)
                lsum = jnp.sum(p, axis=1, keepdims=True)
                ctxh = lax.dot_general(p.astype(jnp.bfloat16), vt_ref[h],
                                       (((1,), (0,)), ((), ())),
                                       preferred_element_type=jnp.float32)
                ctx_buf[pl.ds(r0, CHUNK), h, :] = ctxh
                ml_buf[pl.ds(r0, CHUNK), h:h + 1] = m
                ml_buf[pl.ds(r0, CHUNK), HQ + h:HQ + h + 1] = lsum

        def merge_recv(r0, slot):
            rs = pl.ds(r0, CHUNK)
            m_loc = ml_buf[rs, 0:HQ]
            l_loc = ml_buf[rs, HQ:2 * HQ]
            m_in = recv_ml[slot, :, 0:HQ]
            l_in = recv_ml[slot, :, HQ:2 * HQ]
            m_new = jnp.maximum(m_loc, m_in)
            a = jnp.exp(m_loc - m_new)
            b = jnp.exp(m_in - m_new)
            ml_buf[rs, 0:HQ] = m_new
            ml_buf[rs, HQ:2 * HQ] = a * l_loc + b * l_in
            ctx_buf[rs, :, :] = (a[:, :, None] * ctx_buf[rs, :, :]
                                 + b[:, :, None] * recv_ctx[slot])

        pending = []
        for t in range(N_DEV):
            c = lax.rem(my - t + N_DEV, N_DEV)
            r0 = c * CHUNK
            compute_chunk(r0)
            if t > 0:
                pending[2 * (t - 1)].wait_recv()
                pending[2 * (t - 1) + 1].wait_recv()
                merge_recv(r0, t - 1)
            if t < N_DEV - 1:
                rc = pltpu.make_async_remote_copy(
                    src_ref=ctx_buf.at[pl.ds(r0, CHUNK)],
                    dst_ref=recv_ctx.at[t],
                    send_sem=rs_send.at[t], recv_sem=rs_recv.at[t],
                    device_id=(right,), device_id_type=pl.DeviceIdType.MESH)
                rm = pltpu.make_async_remote_copy(
                    src_ref=ml_buf.at[pl.ds(r0, CHUNK)],
                    dst_ref=recv_ml.at[t],
                    send_sem=ml_send.at[t], recv_sem=ml_recv.at[t],
                    device_id=(right,), device_id_type=pl.DeviceIdType.MESH)
                rc.start()
                rm.start()
                pending.extend([rc, rm])

        own = lax.rem(my + 1, N_DEV)
        r0o = own * CHUNK
        rso = pl.ds(r0o, CHUNK)
        l_fin = ml_buf[rso, HQ:2 * HQ]
        acc = jnp.zeros((CHUNK, DM), jnp.float32)
        for h in range(HQ):
            ch = (ctx_buf[rso, h, :] / l_fin[:, h:h + 1]).astype(jnp.bfloat16)
            acc = acc + lax.dot_general(
                ch, wo_ref[h * DH:(h + 1) * DH, :], (((1,), (0,)), ((), ())),
                preferred_element_type=jnp.float32)
        out_ref[0, rso, :] = acc

        N_R, N_L = 3, N_DEV - 1 - 3
        prev_r = prev_l = None
        for u in range(max(N_R, N_L)):
            if u < N_R:
                g = lax.rem(my + 1 - u + N_DEV, N_DEV)
                rg = pl.ds(g * CHUNK, CHUNK)
                if prev_r is not None:
                    prev_r.wait_recv()
                agr = pltpu.make_async_remote_copy(
                    src_ref=out_ref.at[0, rg], dst_ref=out_ref.at[0, rg],
                    send_sem=ag_send.at[u], recv_sem=ag_recv.at[u],
                    device_id=(right,), device_id_type=pl.DeviceIdType.MESH)
                agr.start()
                pending.append(agr)
                prev_r = agr
            if u < N_L:
                g = lax.rem(my + 1 + u, N_DEV)
                rg = pl.ds(g * CHUNK, CHUNK)
                if prev_l is not None:
                    prev_l.wait_recv()
                agl = pltpu.make_async_remote_copy(
                    src_ref=out_ref.at[0, rg], dst_ref=out_ref.at[0, rg],
                    send_sem=agl_send.at[u], recv_sem=agl_recv.at[u],
                    device_id=(left,), device_id_type=pl.DeviceIdType.MESH)
                agl.start()
                pending.append(agl)
                prev_l = agl
        prev_r.wait_recv()
        prev_l.wait_recv()

        for r in pending:
            r.wait_send()

    call = pl.pallas_call(
        body,
        out_shape=jax.ShapeDtypeStruct((1, SQ, DM), jnp.float32),
        in_specs=[pl.BlockSpec(memory_space=pltpu.VMEM)] * 5,
        out_specs=pl.BlockSpec(memory_space=pltpu.VMEM),
        scratch_shapes=[
            pltpu.VMEM((SQ, DM), jnp.bfloat16),
            pltpu.VMEM((SQ, HQ, DH), jnp.float32),
            pltpu.VMEM((SQ, 2 * HQ), jnp.float32),
            pltpu.VMEM((N_DEV - 1, CHUNK, HQ, DH), jnp.float32),
            pltpu.VMEM((N_DEV - 1, CHUNK, 2 * HQ), jnp.float32),
            pltpu.SemaphoreType.DMA((N_DEV - 1,)),
            pltpu.SemaphoreType.DMA((N_DEV - 1,)),
            pltpu.SemaphoreType.DMA((N_DEV - 1,)),
            pltpu.SemaphoreType.DMA((N_DEV - 1,)),
            pltpu.SemaphoreType.DMA((N_DEV - 1,)),
            pltpu.SemaphoreType.DMA((N_DEV - 1,)),
            pltpu.SemaphoreType.DMA((N_DEV - 1,)),
            pltpu.SemaphoreType.DMA((N_DEV - 1,)),
        ],
        compiler_params=pltpu.CompilerParams(
            collective_id=0, vmem_limit_bytes=63 * 1024 * 1024),
    )
    kt = jnp.transpose(K_ext[0].astype(jnp.bfloat16), (1, 0, 2))
    vt = jnp.transpose(V_ext[0].astype(jnp.bfloat16), (1, 0, 2))
    return call(x.astype(jnp.bfloat16), Wq.astype(jnp.bfloat16),
                kt, vt, Wo.astype(jnp.bfloat16))
